# baseline (speedup 1.0000x reference)
import dataclasses
import functools

import jax
import jax.numpy as jnp
import numpy as np
from jax import lax
from jax.experimental import pallas as pl
from jax.experimental.pallas import tpu as pltpu
from jax.experimental.pallas import tpu_sc as plsc

B, C, H, W = 4, 96, 512, 512
HW = H * W
CB = 16
NCB = C // CB
NSPLIT = 16
CSUB = CB // NSPLIT
RB = 8
NRB = (B * C) // RB

NBINS = 256
SC_CORES, SC_SUBCORES, SC_LANES = 2, 16, 16
NW = SC_CORES * SC_SUBCORES
CHUNK = HW // NW

_C1 = np.float32(256.0 / 255.0)
_C2 = np.float32(255.0 / 256.0)


RS = 32


def _accum_body(*refs):
    x_refs = refs[:NSPLIT]
    stats_ref, bidx_ref, entro_ref, xn_ref = refs[NSPLIT:]
    b = pl.program_id(0)
    cs = pl.program_id(1)
    ws = [jnp.sum(x_refs[k][0, j]) * np.float32(1.0 / (HW * C))
          for k in range(NSPLIT) for j in range(CSUB)]

    def _accumulate(update):
        @pl.loop(0, H, step=RS)
        def _(r):
            sl = pl.ds(r, RS)
            acc = x_refs[0][0, 0, sl, :] * ws[0]
            for k in range(NSPLIT):
                for j in range(CSUB):
                    if k == 0 and j == 0:
                        continue
                    acc += x_refs[k][0, j, sl, :] * ws[k * CSUB + j]
            if update:
                xn_ref[sl, :] += acc
            else:
                xn_ref[sl, :] = acc

    @pl.when(cs == 0)
    def _():
        _accumulate(False)

    @pl.when(cs != 0)
    def _():
        _accumulate(True)

    @pl.when(cs == NCB - 1)
    def _():
        xnb = xn_ref[...]

        @pl.when(b == 0)
        def _():
            entro_ref[...] = xnb * np.float32(1.0 / B)
            stats_ref[...] = jnp.zeros((8, 128), jnp.float32)

        @pl.when(b != 0)
        def _():
            entro_ref[...] += xnb * np.float32(1.0 / B)

        sig = jnp.sum(jax.nn.sigmoid(xnb))
        stats_ref[0:1, :] += sig

        @pl.when(b == B - 1)
        def _():
            e = entro_ref[...]
            mn = jnp.min(e)
            mx = jnp.max(e)
            vn = (e - mn) * (np.float32(255.0) / (mx - mn))
            q = vn * _C1
            i0 = jnp.clip(q.astype(jnp.int32), 0, NBINS - 1)
            e0 = i0.astype(jnp.float32) * _C2
            e1 = (i0 + 1).astype(jnp.float32) * _C2
            i1 = (i0 - jnp.where(vn < e0, 1, 0)
                  + jnp.where(vn >= e1, 1, 0))
            i1 = jnp.clip(i1, 0, NBINS - 1)
            lane = jax.lax.broadcasted_iota(
                jnp.int32, (H, W), 1) & (SC_LANES - 1)
            bidx_ref[...] = i1 * SC_LANES + lane


def _entro_pass(x):
    return pl.pallas_call(
        _accum_body,
        grid=(B, NCB),
        in_specs=[
            pl.BlockSpec((1, CSUB, H, W),
                         functools.partial(
                             lambda k, b, c: (b, c * NSPLIT + k, 0, 0), k))
            for k in range(NSPLIT)
        ],
        out_specs=[
            pl.BlockSpec((8, 128), lambda b, c: (0, 0)),
            pl.BlockSpec((H, W), lambda b, c: (0, 0)),
        ],
        out_shape=[
            jax.ShapeDtypeStruct((8, 128), jnp.float32),
            jax.ShapeDtypeStruct((H, W), jnp.int32),
        ],
        scratch_shapes=[pltpu.VMEM((H, W), jnp.float32),
                        pltpu.VMEM((H, W), jnp.float32)],
    )(*([x] * NSPLIT))


def _sc_compiler_params():
    cp = pltpu.CompilerParams()
    if "needs_layout_passes" in pltpu.CompilerParams.__dataclass_fields__:
        cp = dataclasses.replace(cp, needs_layout_passes=False)
    return cp


def _sc_hist(bidx_flat):
    mesh = plsc.VectorSubcoreMesh(core_axis_name="c", subcore_axis_name="s",
                                  num_cores=SC_CORES, num_subcores=SC_SUBCORES)

    @functools.partial(
        pl.kernel,
        out_type=jax.ShapeDtypeStruct((NW * NBINS * SC_LANES,), jnp.float32),
        mesh=mesh,
        scratch_types=[
            pltpu.VMEM((CHUNK,), jnp.int32),
            pltpu.VMEM((NBINS * SC_LANES,), jnp.float32),
            pltpu.SemaphoreType.DMA,
        ],
        compiler_params=_sc_compiler_params(),
    )
    def hist_kernel(idx_hbm, out_hbm, iv, hist, sem):
        wid = lax.axis_index("s") * SC_CORES + lax.axis_index("c")
        base = wid * CHUNK
        in_dma = pltpu.async_copy(idx_hbm.at[pl.ds(base, CHUNK)], iv, sem)

        ones = jnp.ones((SC_LANES,), jnp.float32)
        zeros_f = jnp.zeros((SC_LANES,), jnp.float32)

        @pl.loop(0, NBINS * SC_LANES, step=8 * SC_LANES)
        def _(j):
            for u in range(8):
                hist[pl.ds(j + u * SC_LANES, SC_LANES)] = zeros_f

        in_dma.wait()

        @pl.loop(0, CHUNK, step=8 * SC_LANES)
        def _(i):
            for u in range(8):
                plsc.addupdate_scatter(
                    hist, [iv[pl.ds(i + u * SC_LANES, SC_LANES)]], ones)

        pltpu.sync_copy(
            hist, out_hbm.at[pl.ds(wid * NBINS * SC_LANES, NBINS * SC_LANES)])

    return hist_kernel(bidx_flat)


def _final_body(h_ref, st_ref, o_ref):
    counts = jnp.sum(h_ref[...], axis=(0, 2))
    his = counts * np.float32(1.0 / HW)
    ent = jnp.sum(his * -jnp.log(his + np.float32(1e-8)))
    nnz = jnp.sum(jnp.where(counts > 0,
                            jnp.float32(1.0), jnp.float32(0.0)))
    entro_final = ent / nnz
    xs = st_ref[0, 0] * np.float32(1.0 / (B * HW))
    o_ref[...] = jnp.broadcast_to(
        (xs + entro_final * np.float32(10.0)).reshape(1, 1), (8, 128))


def _final(hists, stats):
    return pl.pallas_call(
        _final_body,
        in_specs=[
            pl.BlockSpec((NW, NBINS, SC_LANES), lambda: (0, 0, 0)),
            pl.BlockSpec((8, 128), lambda: (0, 0)),
        ],
        out_specs=pl.BlockSpec((8, 128), lambda: (0, 0)),
        out_shape=jax.ShapeDtypeStruct((8, 128), jnp.float32),
    )(hists, stats)


def kernel(x):
    stats, bidx = _entro_pass(x)
    hists = _sc_hist(bidx.reshape(HW))
    res = _final(hists.reshape(NW, NBINS, SC_LANES), stats)
    return res[0, 0]


# speedup vs baseline: 1.0455x; 1.0455x over previous
import dataclasses
import functools

import jax
import jax.numpy as jnp
import numpy as np
from jax import lax
from jax.experimental import pallas as pl
from jax.experimental.pallas import tpu as pltpu
from jax.experimental.pallas import tpu_sc as plsc

B, C, H, W = 4, 96, 512, 512
HW = H * W
CB = 16
NCB = C // CB
NSPLIT = 16
CSUB = CB // NSPLIT
RB = 8
NRB = (B * C) // RB

NBINS = 256
SC_CORES, SC_SUBCORES, SC_LANES = 1, 16, 16
NW = SC_CORES * SC_SUBCORES
CHUNK = HW // NW

_C1 = np.float32(256.0 / 255.0)
_C2 = np.float32(255.0 / 256.0)


RS = 32


def _accum_body(*refs):
    x_refs = refs[:NSPLIT]
    stats_ref, bidx_ref, entro_ref, xn_ref = refs[NSPLIT:]
    b = pl.program_id(0)
    cs = pl.program_id(1)
    ws = [jnp.sum(x_refs[k][0, j]) * np.float32(1.0 / (HW * C))
          for k in range(NSPLIT) for j in range(CSUB)]

    def _accumulate(update):
        @pl.loop(0, H, step=RS)
        def _(r):
            sl = pl.ds(r, RS)
            acc = x_refs[0][0, 0, sl, :] * ws[0]
            for k in range(NSPLIT):
                for j in range(CSUB):
                    if k == 0 and j == 0:
                        continue
                    acc += x_refs[k][0, j, sl, :] * ws[k * CSUB + j]
            if update:
                xn_ref[sl, :] += acc
            else:
                xn_ref[sl, :] = acc

    @pl.when(cs == 0)
    def _():
        _accumulate(False)

    @pl.when(cs != 0)
    def _():
        _accumulate(True)

    @pl.when(cs == NCB - 1)
    def _():
        xnb = xn_ref[...]

        @pl.when(b == 0)
        def _():
            entro_ref[...] = xnb * np.float32(1.0 / B)
            stats_ref[...] = jnp.zeros((8, 128), jnp.float32)

        @pl.when(b != 0)
        def _():
            entro_ref[...] += xnb * np.float32(1.0 / B)

        sig = jnp.sum(jax.nn.sigmoid(xnb))
        stats_ref[0:1, :] += sig

        @pl.when(b == B - 1)
        def _():
            e = entro_ref[...]
            mn = jnp.min(e)
            mx = jnp.max(e)
            vn = (e - mn) * (np.float32(255.0) / (mx - mn))
            q = vn * _C1
            i0 = jnp.clip(q.astype(jnp.int32), 0, NBINS - 1)
            e0 = i0.astype(jnp.float32) * _C2
            e1 = (i0 + 1).astype(jnp.float32) * _C2
            i1 = (i0 - jnp.where(vn < e0, 1, 0)
                  + jnp.where(vn >= e1, 1, 0))
            i1 = jnp.clip(i1, 0, NBINS - 1)
            lane = jax.lax.broadcasted_iota(
                jnp.int32, (H, W), 1) & (SC_LANES - 1)
            bidx_ref[...] = i1 * SC_LANES + lane


def _entro_pass(x):
    return pl.pallas_call(
        _accum_body,
        grid=(B, NCB),
        in_specs=[
            pl.BlockSpec((1, CSUB, H, W),
                         functools.partial(
                             lambda k, b, c: (b, c * NSPLIT + k, 0, 0), k))
            for k in range(NSPLIT)
        ],
        out_specs=[
            pl.BlockSpec((8, 128), lambda b, c: (0, 0)),
            pl.BlockSpec((H, W), lambda b, c: (0, 0)),
        ],
        out_shape=[
            jax.ShapeDtypeStruct((8, 128), jnp.float32),
            jax.ShapeDtypeStruct((H, W), jnp.int32),
        ],
        scratch_shapes=[pltpu.VMEM((H, W), jnp.float32),
                        pltpu.VMEM((H, W), jnp.float32)],
    )(*([x] * NSPLIT))


def _sc_compiler_params():
    cp = pltpu.CompilerParams()
    if "needs_layout_passes" in pltpu.CompilerParams.__dataclass_fields__:
        cp = dataclasses.replace(cp, needs_layout_passes=False)
    return cp


def _sc_hist(bidx_flat):
    mesh = plsc.VectorSubcoreMesh(core_axis_name="c", subcore_axis_name="s",
                                  num_cores=SC_CORES, num_subcores=SC_SUBCORES)

    @functools.partial(
        pl.kernel,
        out_type=jax.ShapeDtypeStruct((NW * NBINS * SC_LANES,), jnp.float32),
        mesh=mesh,
        scratch_types=[
            pltpu.VMEM((CHUNK,), jnp.int32),
            pltpu.VMEM((NBINS * SC_LANES,), jnp.float32),
            pltpu.SemaphoreType.DMA,
        ],
        compiler_params=_sc_compiler_params(),
    )
    def hist_kernel(idx_hbm, out_hbm, iv, hist, sem):
        wid = lax.axis_index("s") * SC_CORES + lax.axis_index("c")
        base = wid * CHUNK
        in_dma = pltpu.async_copy(idx_hbm.at[pl.ds(base, CHUNK)], iv, sem)

        ones = jnp.ones((SC_LANES,), jnp.float32)
        zeros_f = jnp.zeros((SC_LANES,), jnp.float32)

        @pl.loop(0, NBINS * SC_LANES, step=8 * SC_LANES)
        def _(j):
            for u in range(8):
                hist[pl.ds(j + u * SC_LANES, SC_LANES)] = zeros_f

        in_dma.wait()

        @pl.loop(0, CHUNK, step=8 * SC_LANES)
        def _(i):
            for u in range(8):
                plsc.addupdate_scatter(
                    hist, [iv[pl.ds(i + u * SC_LANES, SC_LANES)]], ones)

        pltpu.sync_copy(
            hist, out_hbm.at[pl.ds(wid * NBINS * SC_LANES, NBINS * SC_LANES)])

    return hist_kernel(bidx_flat)


def _final_body(h_ref, st_ref, o_ref):
    counts = jnp.sum(h_ref[...], axis=(0, 2))
    his = counts * np.float32(1.0 / HW)
    ent = jnp.sum(his * -jnp.log(his + np.float32(1e-8)))
    nnz = jnp.sum(jnp.where(counts > 0,
                            jnp.float32(1.0), jnp.float32(0.0)))
    entro_final = ent / nnz
    xs = st_ref[0, 0] * np.float32(1.0 / (B * HW))
    o_ref[...] = jnp.broadcast_to(
        (xs + entro_final * np.float32(10.0)).reshape(1, 1), (8, 128))


def _final(hists, stats):
    return pl.pallas_call(
        _final_body,
        in_specs=[
            pl.BlockSpec((NW, NBINS, SC_LANES), lambda: (0, 0, 0)),
            pl.BlockSpec((8, 128), lambda: (0, 0)),
        ],
        out_specs=pl.BlockSpec((8, 128), lambda: (0, 0)),
        out_shape=jax.ShapeDtypeStruct((8, 128), jnp.float32),
    )(hists, stats)


def kernel(x):
    stats, bidx = _entro_pass(x)
    hists = _sc_hist(bidx.reshape(HW))
    res = _final(hists.reshape(NW, NBINS, SC_LANES), stats)
    return res[0, 0]
